# baseline (speedup 1.0000x reference)
import jax
import jax.numpy as jnp
from jax import lax
from jax.experimental import pallas as pl
from jax.experimental.pallas import tpu as pltpu
from jax.experimental.pallas import tpu_sc as plsc

_BATCH = 16384
_LEN = 200
_VOCAB = 20000
_NTILES = 16
_COLS_PER_TILE = _BATCH // _NTILES
_CCOLS = 128
_NCHUNK = _COLS_PER_TILE // _CCOLS
_NBUF = 2
_LANES = 16
_SLICES = _CCOLS // _LANES


def _body(src_hbm, tgt_hbm, srctab_hbm, tgttab_hbm, src_out, tgt_out,
          tab_v, *rest):
    ids_bufs = rest[0:_NBUF]
    out_bufs = rest[_NBUF:2 * _NBUF]
    si = rest[2 * _NBUF:3 * _NBUF]
    so = rest[3 * _NBUF:4 * _NBUF]
    c = lax.axis_index("c")
    s = lax.axis_index("s")

    def do_side(ids_hbm, tab_hbm, out_hbm):
        col0 = s * _COLS_PER_TILE

        for b in range(_NBUF):
            pltpu.async_copy(
                ids_hbm.at[:, pl.ds(col0 + b * _CCOLS, _CCOLS)],
                ids_bufs[b], si[b],
            )
        pltpu.sync_copy(tab_hbm, tab_v)

        def outer(i, carry):
            for b in range(_NBUF):
                ids_v, out_v, sem_i, sem_o = (
                    ids_bufs[b], out_bufs[b], si[b], so[b]
                )
                kk = i * _NBUF + b
                base = col0 + kk * _CCOLS

                pltpu.make_async_copy(
                    ids_hbm.at[:, pl.ds(base, _CCOLS)], ids_v, sem_i
                ).wait()

                @pl.when(kk >= _NBUF)
                def _():
                    pltpu.make_async_copy(
                        out_v, out_hbm.at[:, pl.ds(base, _CCOLS)], sem_o
                    ).wait()

                def gather_row(r, carry2):
                    idxs = [
                        ids_v[r, pl.ds(t * _LANES, _LANES)]
                        for t in range(_SLICES)
                    ]
                    vals = [plsc.load_gather(tab_v, [i]) for i in idxs]
                    for t, val in enumerate(vals):
                        out_v[r, pl.ds(t * _LANES, _LANES)] = val
                    return carry2

                lax.fori_loop(0, _LEN, gather_row, 0)

                pltpu.async_copy(
                    out_v, out_hbm.at[:, pl.ds(base, _CCOLS)], sem_o
                )

                @pl.when(kk + _NBUF < _NCHUNK)
                def _():
                    pltpu.async_copy(
                        ids_hbm.at[:, pl.ds(base + _NBUF * _CCOLS, _CCOLS)],
                        ids_v,
                        sem_i,
                    )

            return carry

        lax.fori_loop(0, _NCHUNK // _NBUF, outer, 0)

        for b in range(_NBUF):
            pltpu.make_async_copy(
                out_bufs[b], out_hbm.at[:, pl.ds(col0, _CCOLS)], so[b]
            ).wait()

    @pl.when(c == 0)
    def _():
        do_side(src_hbm, srctab_hbm, src_out)

    @pl.when(c == 1)
    def _():
        do_side(tgt_hbm, tgttab_hbm, tgt_out)


def kernel(src_ids, tgt_ids, src_table, tgt_table):
    mesh = plsc.VectorSubcoreMesh(core_axis_name="c", subcore_axis_name="s")
    f = pl.kernel(
        _body,
        mesh=mesh,
        out_type=(
            jax.ShapeDtypeStruct((_LEN, _BATCH), jnp.float32),
            jax.ShapeDtypeStruct((_LEN, _BATCH), jnp.float32),
        ),
        scratch_types=(
            [pltpu.VMEM((_VOCAB,), jnp.float32)]
            + [pltpu.VMEM((_LEN, _CCOLS), jnp.int32) for _ in range(_NBUF)]
            + [pltpu.VMEM((_LEN, _CCOLS), jnp.float32) for _ in range(_NBUF)]
            + [pltpu.SemaphoreType.DMA for _ in range(2 * _NBUF)]
        ),
        compiler_params=pltpu.CompilerParams(needs_layout_passes=False),
    )
    src_t, tgt_t = f(src_ids.T, tgt_ids.T, src_table, tgt_table)
    return (src_t.T, tgt_t.T)
